# baseline (speedup 1.0000x reference)
import functools
import math

import jax
import jax.numpy as jnp
from jax import lax
from jax.experimental import pallas as pl
from jax.experimental.pallas import tpu as pltpu
from jax.experimental.pallas import tpu_sc as plsc

_N_POS = 1000000
_D = 64
_NB = math.ceil(math.log2(_N_POS))
_LO = 10
_HI = _NB - _LO
_T2 = (1 << _LO) + (1 << _HI)

_NC = 2
_NS = 16
_NW = _NC * _NS
_L = 16

_CHUNK = 128


def _t2_body(tlo_ref, thi_ref, out_ref):
    n = 1 << _LO
    v = lax.broadcasted_iota(jnp.int32, (n, 32), 0)
    b = lax.broadcasted_iota(jnp.int32, (n, 32), 1)
    bits = ((v >> b) & 1).astype(jnp.float32)
    out_ref[:n] = lax.dot(bits, tlo_ref[...],
                          precision=lax.Precision.HIGHEST)
    out_ref[n:] = lax.dot(bits, thi_ref[...],
                          precision=lax.Precision.HIGHEST)


def _build_table2(table, interpret=False):
    tlo = jnp.zeros((32, _D), jnp.float32).at[:_LO].set(table[:_LO])
    thi = jnp.zeros((32, _D), jnp.float32).at[:_HI].set(table[_LO:_NB])
    return pl.pallas_call(
        _t2_body,
        out_shape=jax.ShapeDtypeStruct((_T2, _D), jnp.float32),
        interpret=interpret,
    )(tlo, thi)


def _sc_embed(x_flat, t2):
    t = x_flat.shape[0]
    per_w = t // _NW
    n_pairs = per_w // (2 * _CHUNK)
    mesh = plsc.VectorSubcoreMesh(core_axis_name="c", subcore_axis_name="s")

    nbuf = 4
    n_steps = per_w // (nbuf * _CHUNK)
    idx_types = []
    for _i in range(nbuf):
        idx_types += [pltpu.VMEM((_CHUNK,), jnp.int32),
                      pltpu.VMEM((_CHUNK,), jnp.int32)]
    buf_types = [pltpu.VMEM((_CHUNK, _D), jnp.float32) for _i in range(nbuf)]

    @functools.partial(
        pl.kernel, mesh=mesh,
        out_type=jax.ShapeDtypeStruct((t, _D), jnp.float32),
        scratch_types=(
            [pltpu.VMEM((per_w,), jnp.int32)] + idx_types + buf_types
            + [pltpu.VMEM_SHARED((_T2, _D), jnp.float32),
               pltpu.SemaphoreType.DMA,
               pltpu.SemaphoreType.DMA,
               pltpu.SemaphoreType.DMA]
        ),
    )
    def k(x_hbm, t2_hbm, out_hbm, x_v, *rest):
        idx_refs = rest[:2 * nbuf]
        bufs = rest[2 * nbuf:3 * nbuf]
        t2_spm, gs_a, gs_b, ws = rest[3 * nbuf:]
        wid = lax.axis_index("s") * _NC + lax.axis_index("c")
        sid = lax.axis_index("s")
        base = wid * per_w
        @pl.when(sid == 0)
        def _():
            pltpu.sync_copy(t2_hbm, t2_spm)
        plsc.subcore_barrier()
        pltpu.sync_copy(x_hbm.at[pl.ds(base, per_w)], x_v)

        def compute_idx(off, ilo, ihi):
            for i in range(_CHUNK // _L):
                v = x_v[pl.ds(off + i * _L, _L)]
                ilo[pl.ds(i * _L, _L)] = v & ((1 << _LO) - 1)
                ihi[pl.ds(i * _L, _L)] = (v >> _LO) + (1 << _LO)

        def step(b, _):
            j0 = b * 2
            off0 = j0 * _CHUNK
            off1 = (j0 + 1) * _CHUNK
            compute_idx(off0, idx_refs[0], idx_refs[1])
            compute_idx(off1, idx_refs[2], idx_refs[3])
            g1a = pltpu.async_copy(t2_spm.at[idx_refs[0]], bufs[0], gs_a)
            g1b = pltpu.async_copy(t2_spm.at[idx_refs[2]], bufs[1], gs_b)
            g1a.wait()
            g2a = pltpu.async_copy(t2_spm.at[idx_refs[1]], bufs[0], gs_a,
                                   add=True)
            g1b.wait()
            g2b = pltpu.async_copy(t2_spm.at[idx_refs[3]], bufs[1], gs_b,
                                   add=True)
            g2a.wait()
            w0 = pltpu.async_copy(
                bufs[0], out_hbm.at[pl.ds(base + off0, _CHUNK), :], ws)
            g2b.wait()
            w1 = pltpu.async_copy(
                bufs[1], out_hbm.at[pl.ds(base + off1, _CHUNK), :], ws)
            w0.wait()
            w1.wait()
            return 0

        lax.fori_loop(0, per_w // (2 * _CHUNK), step, 0)

    return k(x_flat, t2)


def kernel(x, table):
    x_flat = x.reshape(-1)
    t2 = _build_table2(table)
    return _sc_embed(x_flat, t2)
